# baseline (speedup 1.0000x reference)
import functools

import jax
import jax.numpy as jnp
from jax import lax
from jax.experimental import pallas as pl
from jax.experimental.pallas import tpu as pltpu
from jax.experimental.pallas import tpu_sc as plsc

N = 10000
D = 128
E = 320000

NC = 2
NS = 16
NW = NC * NS
CH = 96
EPT = 10080
NCHUNK = EPT // CH
EPAD = NW * EPT
NPAD = 10240
TRASH = N
RPT = NPAD // NS


RING = 3
AHEAD = RING - 1
NITER = -(-NCHUNK // RING)
LANES = 16
SHIFT = 14


@functools.lru_cache(maxsize=None)
def _make_spmm(F: int):
    mesh = plsc.VectorSubcoreMesh(core_axis_name="c", subcore_axis_name="s")

    scratch = (
        [pltpu.VMEM((EPT,), jnp.int32)]
        + [pltpu.VMEM((CH,), jnp.int32) for _ in range(RING)]
        + [pltpu.VMEM((CH,), jnp.int32) for _ in range(RING)]
        + [pltpu.VMEM((CH, F), jnp.float32) for _ in range(RING)]
        + [pltpu.VMEM_SHARED((NPAD, F), jnp.float32)]
        + [pltpu.SemaphoreType.DMA for _ in range(2 * RING)]
    )

    @functools.partial(
        pl.kernel,
        out_type=jax.ShapeDtypeStruct((NC, NPAD, F), jnp.float32),
        mesh=mesh,
        scratch_types=scratch,
    )
    def spmm(support_hbm, edges_hbm, zeros_hbm, out_hbm, *scr):
        idx_all = scr[0]
        src_v = scr[1:1 + RING]
        dst_v = scr[1 + RING:1 + 2 * RING]
        rows = scr[1 + 2 * RING:1 + 3 * RING]
        acc_sh = scr[1 + 3 * RING]
        g_sem = scr[2 + 3 * RING:2 + 4 * RING]
        s_sem = scr[2 + 4 * RING:2 + 5 * RING]

        cid = lax.axis_index("c")
        sid = lax.axis_index("s")
        wid = sid * NC + cid

        pltpu.async_copy(edges_hbm.at[pl.ds(wid * EPT, EPT)], idx_all,
                         g_sem[0])
        pltpu.sync_copy(zeros_hbm.at[pl.ds(sid * RPT, RPT)],
                        acc_sh.at[pl.ds(sid * RPT, RPT)])
        pltpu.make_async_copy(edges_hbm.at[pl.ds(wid * EPT, EPT)], idx_all,
                              g_sem[0]).wait()
        plsc.subcore_barrier()

        def unpack_and_gather(c, b):
            base = pl.multiple_of(c * CH, LANES)
            for i in range(CH // LANES):
                p = idx_all[pl.ds(base + i * LANES, LANES)]
                src_v[b][pl.ds(i * LANES, LANES)] = p & ((1 << SHIFT) - 1)
                dst_v[b][pl.ds(i * LANES, LANES)] = lax.shift_right_logical(
                    p, SHIFT)
            pltpu.async_copy(support_hbm.at[src_v[b]], rows[b], g_sem[b])

        for c in range(AHEAD):
            unpack_and_gather(c, c)

        def body(j, carry):
            for k in range(RING):
                g = j * RING + k
                bc = (k + AHEAD) % RING
                c = g + AHEAD

                @pl.when(c < NCHUNK)
                def _prefetch():
                    @pl.when(c >= RING)
                    def _drain():
                        pltpu.make_async_copy(
                            rows[bc], acc_sh.at[dst_v[bc]], s_sem[bc]).wait()
                    unpack_and_gather(c, bc)

                @pl.when(g < NCHUNK)
                def _consume():
                    pltpu.make_async_copy(
                        support_hbm.at[src_v[k]], rows[k], g_sem[k]).wait()
                    pltpu.async_copy(rows[k], acc_sh.at[dst_v[k]], s_sem[k],
                                     add=True)
            return carry

        lax.fori_loop(0, NITER, body, 0)

        for q in range(NCHUNK - RING, NCHUNK):
            b = q % RING
            pltpu.make_async_copy(rows[b], acc_sh.at[dst_v[b]], s_sem[b]).wait()
        plsc.subcore_barrier()

        pltpu.sync_copy(acc_sh.at[pl.ds(sid * RPT, RPT)],
                        out_hbm.at[cid, pl.ds(sid * RPT, RPT)])

    return spmm


def _spmm(support, edges, zeros):
    return _make_spmm(support.shape[1])(support, edges, zeros)


BM = 2000


def _mm2_body(x_ref, w1_ref, w2_ref, o1_ref, o2_ref):
    x = x_ref[...]
    o1_ref[...] = jnp.dot(x, w1_ref[...], preferred_element_type=jnp.float32)
    o2_ref[...] = jnp.dot(x, w2_ref[...], preferred_element_type=jnp.float32)


def _mm2(x, w1, w2):
    m, k = x.shape
    f = w1.shape[1]
    return pl.pallas_call(
        _mm2_body,
        grid=(m // BM,),
        in_specs=[pl.BlockSpec((BM, k), lambda i: (i, 0)),
                  pl.BlockSpec((k, f), lambda i: (0, 0)),
                  pl.BlockSpec((k, f), lambda i: (0, 0))],
        out_specs=[pl.BlockSpec((BM, f), lambda i: (i, 0)),
                   pl.BlockSpec((BM, f), lambda i: (i, 0))],
        out_shape=[jax.ShapeDtypeStruct((m, f), jnp.float32),
                   jax.ShapeDtypeStruct((m, f), jnp.float32)],
    )(x, w1, w2)


def _relu_mm_body(p_ref, b_ref, w_ref, o_ref):
    h = jnp.maximum(p_ref[0] + p_ref[1] + b_ref[...], 0.0)
    o_ref[...] = jnp.dot(h, w_ref[...], preferred_element_type=jnp.float32)


def _relu_mm(p, b, w):
    f = p.shape[2]
    f2 = w.shape[1]
    return pl.pallas_call(
        _relu_mm_body,
        grid=(N // BM,),
        in_specs=[pl.BlockSpec((NC, BM, f), lambda i: (0, i, 0)),
                  pl.BlockSpec((1, f), lambda i: (0, 0)),
                  pl.BlockSpec((f, f2), lambda i: (0, 0))],
        out_specs=pl.BlockSpec((BM, f2), lambda i: (i, 0)),
        out_shape=jax.ShapeDtypeStruct((N, f2), jnp.float32),
    )(p, b.reshape(1, f), w)


def _normalize_rows(v, eps=1e-12):
    n = jnp.sqrt(jnp.sum(v * v, axis=1, keepdims=True))
    return v / jnp.maximum(n, eps)


def _norm_mm_body(p_ref, b_ref, w_ref, o_ref):
    s = _normalize_rows(p_ref[0] + p_ref[1] + b_ref[...])
    o_ref[...] = jnp.dot(s, w_ref[...], preferred_element_type=jnp.float32)


def _norm_mm(p, b, w):
    f = p.shape[2]
    f2 = w.shape[1]
    return pl.pallas_call(
        _norm_mm_body,
        grid=(N // BM,),
        in_specs=[pl.BlockSpec((NC, BM, f), lambda i: (0, i, 0)),
                  pl.BlockSpec((1, f), lambda i: (0, 0)),
                  pl.BlockSpec((f, f2), lambda i: (0, 0))],
        out_specs=pl.BlockSpec((BM, f2), lambda i: (i, 0)),
        out_shape=jax.ShapeDtypeStruct((N, f2), jnp.float32),
    )(p, b.reshape(1, f), w)


def _final_body(x_ref, q0_ref, b0_ref, q1_ref, b1_ref, o_ref):
    x = x_ref[...]
    s0 = _normalize_rows(q0_ref[0] + q0_ref[1] + b0_ref[...])
    s1 = _normalize_rows(q1_ref[0] + q1_ref[1] + b1_ref[...])
    c1 = _normalize_rows(jnp.concatenate([x, s0], axis=1))
    o_ref[...] = _normalize_rows(jnp.concatenate([c1, s1], axis=1))


def _final(x, q0, b0, q1, b1):
    f = D
    return pl.pallas_call(
        _final_body,
        grid=(N // BM,),
        in_specs=[pl.BlockSpec((BM, f), lambda i: (i, 0)),
                  pl.BlockSpec((NC, BM, f), lambda i: (0, i, 0)),
                  pl.BlockSpec((1, f), lambda i: (0, 0)),
                  pl.BlockSpec((NC, BM, f), lambda i: (0, i, 0)),
                  pl.BlockSpec((1, f), lambda i: (0, 0))],
        out_specs=pl.BlockSpec((BM, 3 * f), lambda i: (i, 0)),
        out_shape=jax.ShapeDtypeStruct((N, 3 * f), jnp.float32),
    )(x, q0, b0.reshape(1, f), q1, b1.reshape(1, f))


def kernel(x, edge_index, W1_00, b1_00, W2_00, b2_00, W1_10, b1_10, W2_10,
           b2_10, W1_11, b1_11, W2_11, b2_11):
    packed = edge_index[1] * (1 << SHIFT) + edge_index[0]
    edges = jnp.concatenate(
        [packed, jnp.full((EPAD - E,), TRASH << SHIFT, jnp.int32)])
    zeros = jnp.zeros((NPAD, D), jnp.float32)

    t00, t10 = _mm2(x, W1_00, W1_10)
    a00 = _spmm(t00, edges, zeros)
    a10 = _spmm(t10, edges, zeros)
    t01 = _relu_mm(a00, b1_00, W2_00)
    t11 = _relu_mm(a10, b1_10, W2_10)
    q00 = _spmm(t01, edges, zeros)
    q10 = _spmm(t11, edges, zeros)

    t20 = _norm_mm(q10, b2_10, W1_11)
    a20 = _spmm(t20, edges, zeros)
    t21 = _relu_mm(a20, b1_11, W2_11)
    q11 = _spmm(t21, edges, zeros)

    return _final(x, q00, b2_00, q11, b2_11)


# speedup vs baseline: 2.2599x; 2.2599x over previous
import functools

import jax
import jax.numpy as jnp
from jax import lax
from jax.experimental import pallas as pl
from jax.experimental.pallas import tpu as pltpu
from jax.experimental.pallas import tpu_sc as plsc

N = 10000
D = 128
E = 320000

NC = 2
NS = 16
NW = NC * NS
CH = 96
EPT = 10080
NCHUNK = EPT // CH
EPAD = NW * EPT
NPAD = 10240
TRASH = N
RPT = NPAD // NS


RING = 3
AHEAD = RING - 1
NITER = -(-NCHUNK // RING)
LANES = 16
SHIFT = 14


@functools.lru_cache(maxsize=None)
def _make_spmm(F: int):
    mesh = plsc.VectorSubcoreMesh(core_axis_name="c", subcore_axis_name="s")

    scratch = (
        [pltpu.VMEM((EPT,), jnp.int32)]
        + [pltpu.VMEM((CH,), jnp.int32) for _ in range(RING)]
        + [pltpu.VMEM((CH,), jnp.int32) for _ in range(RING)]
        + [pltpu.VMEM((CH, F), jnp.float32) for _ in range(RING)]
        + [pltpu.VMEM_SHARED((NPAD, F), jnp.float32)]
        + [pltpu.SemaphoreType.DMA for _ in range(2 * RING)]
    )

    @functools.partial(
        pl.kernel,
        out_type=jax.ShapeDtypeStruct((NC, NPAD, F), jnp.float32),
        mesh=mesh,
        scratch_types=scratch,
    )
    def spmm(support_hbm, edges_hbm, zeros_hbm, out_hbm, *scr):
        idx_all = scr[0]
        src_v = scr[1:1 + RING]
        dst_v = scr[1 + RING:1 + 2 * RING]
        rows = scr[1 + 2 * RING:1 + 3 * RING]
        acc_sh = scr[1 + 3 * RING]
        g_sem = scr[2 + 3 * RING:2 + 4 * RING]
        s_sem = scr[2 + 4 * RING:2 + 5 * RING]

        cid = lax.axis_index("c")
        sid = lax.axis_index("s")
        wid = sid * NC + cid

        pltpu.async_copy(edges_hbm.at[pl.ds(wid * EPT, EPT)], idx_all,
                         g_sem[0])
        pltpu.sync_copy(zeros_hbm.at[pl.ds(sid * RPT, RPT)],
                        acc_sh.at[pl.ds(sid * RPT, RPT)])
        pltpu.make_async_copy(edges_hbm.at[pl.ds(wid * EPT, EPT)], idx_all,
                              g_sem[0]).wait()
        plsc.subcore_barrier()

        def unpack_and_gather(c, b):
            base = pl.multiple_of(c * CH, LANES)
            for i in range(CH // LANES):
                p = idx_all[pl.ds(base + i * LANES, LANES)]
                src_v[b][pl.ds(i * LANES, LANES)] = p & ((1 << SHIFT) - 1)
                dst_v[b][pl.ds(i * LANES, LANES)] = lax.shift_right_logical(
                    p, SHIFT)
            pltpu.async_copy(support_hbm.at[src_v[b]], rows[b], g_sem[b])

        for c in range(AHEAD):
            unpack_and_gather(c, c)

        def body(j, carry):
            for k in range(RING):
                g = j * RING + k
                bc = (k + AHEAD) % RING
                c = g + AHEAD

                @pl.when(c < NCHUNK)
                def _prefetch():
                    @pl.when(c >= RING)
                    def _drain():
                        pltpu.make_async_copy(
                            rows[bc], acc_sh.at[dst_v[bc]], s_sem[bc]).wait()
                    unpack_and_gather(c, bc)

                @pl.when(g < NCHUNK)
                def _consume():
                    pltpu.make_async_copy(
                        support_hbm.at[src_v[k]], rows[k], g_sem[k]).wait()
                    pltpu.async_copy(rows[k], acc_sh.at[dst_v[k]], s_sem[k],
                                     add=True)
            return carry

        lax.fori_loop(0, NITER, body, 0)

        for q in range(NCHUNK - RING, NCHUNK):
            b = q % RING
            pltpu.make_async_copy(rows[b], acc_sh.at[dst_v[b]], s_sem[b]).wait()
        plsc.subcore_barrier()

        pltpu.sync_copy(acc_sh.at[pl.ds(sid * RPT, RPT)],
                        out_hbm.at[cid, pl.ds(sid * RPT, RPT)])

    return spmm


def _spmm(support, edges, zeros):
    return _make_spmm(support.shape[1])(support, edges, zeros)


BM = 2000


def _mm2_body(x_ref, w1_ref, w2_ref, o1_ref, o2_ref):
    x = x_ref[...]
    o1_ref[...] = jnp.dot(x, w1_ref[...], preferred_element_type=jnp.float32)
    o2_ref[...] = jnp.dot(x, w2_ref[...], preferred_element_type=jnp.float32)


def _mm2(x, w1, w2):
    m, k = x.shape
    f = w1.shape[1]
    return pl.pallas_call(
        _mm2_body,
        grid=(m // BM,),
        in_specs=[pl.BlockSpec((BM, k), lambda i: (i, 0)),
                  pl.BlockSpec((k, f), lambda i: (0, 0)),
                  pl.BlockSpec((k, f), lambda i: (0, 0))],
        out_specs=[pl.BlockSpec((BM, f), lambda i: (i, 0)),
                   pl.BlockSpec((BM, f), lambda i: (i, 0))],
        out_shape=[jax.ShapeDtypeStruct((m, f), jnp.float32),
                   jax.ShapeDtypeStruct((m, f), jnp.float32)],
    )(x, w1, w2)


def _relu_mm_body(p_ref, b_ref, w_ref, o_ref):
    h = jnp.maximum(p_ref[0] + p_ref[1] + b_ref[...], 0.0)
    o_ref[...] = jnp.dot(h, w_ref[...], preferred_element_type=jnp.float32)


def _relu_mm(p, b, w):
    f = p.shape[2]
    f2 = w.shape[1]
    return pl.pallas_call(
        _relu_mm_body,
        grid=(N // BM,),
        in_specs=[pl.BlockSpec((NC, BM, f), lambda i: (0, i, 0)),
                  pl.BlockSpec((1, f), lambda i: (0, 0)),
                  pl.BlockSpec((f, f2), lambda i: (0, 0))],
        out_specs=pl.BlockSpec((BM, f2), lambda i: (i, 0)),
        out_shape=jax.ShapeDtypeStruct((N, f2), jnp.float32),
    )(p, b.reshape(1, f), w)


def _normalize_rows(v, eps=1e-12):
    n = jnp.sqrt(jnp.sum(v * v, axis=1, keepdims=True))
    return v / jnp.maximum(n, eps)


def _norm_mm_body(p_ref, b_ref, w_ref, o_ref):
    s = _normalize_rows(p_ref[0] + p_ref[1] + b_ref[...])
    o_ref[...] = jnp.dot(s, w_ref[...], preferred_element_type=jnp.float32)


def _norm_mm(p, b, w):
    f = p.shape[2]
    f2 = w.shape[1]
    return pl.pallas_call(
        _norm_mm_body,
        grid=(N // BM,),
        in_specs=[pl.BlockSpec((NC, BM, f), lambda i: (0, i, 0)),
                  pl.BlockSpec((1, f), lambda i: (0, 0)),
                  pl.BlockSpec((f, f2), lambda i: (0, 0))],
        out_specs=pl.BlockSpec((BM, f2), lambda i: (i, 0)),
        out_shape=jax.ShapeDtypeStruct((N, f2), jnp.float32),
    )(p, b.reshape(1, f), w)


def _final_body(x_ref, q0_ref, b0_ref, q1_ref, b1_ref, o_ref):
    x = x_ref[...]
    s0 = _normalize_rows(q0_ref[0] + q0_ref[1] + b0_ref[...])
    s1 = _normalize_rows(q1_ref[0] + q1_ref[1] + b1_ref[...])
    c1 = _normalize_rows(jnp.concatenate([x, s0], axis=1))
    o_ref[...] = _normalize_rows(jnp.concatenate([c1, s1], axis=1))


def _final(x, q0, b0, q1, b1):
    f = D
    return pl.pallas_call(
        _final_body,
        grid=(N // BM,),
        in_specs=[pl.BlockSpec((BM, f), lambda i: (i, 0)),
                  pl.BlockSpec((NC, BM, f), lambda i: (0, i, 0)),
                  pl.BlockSpec((1, f), lambda i: (0, 0)),
                  pl.BlockSpec((NC, BM, f), lambda i: (0, i, 0)),
                  pl.BlockSpec((1, f), lambda i: (0, 0))],
        out_specs=pl.BlockSpec((BM, 3 * f), lambda i: (i, 0)),
        out_shape=jax.ShapeDtypeStruct((N, 3 * f), jnp.float32),
    )(x, q0, b0.reshape(1, f), q1, b1.reshape(1, f))


def kernel(x, edge_index, W1_00, b1_00, W2_00, b2_00, W1_10, b1_10, W2_10,
           b2_10, W1_11, b1_11, W2_11, b2_11):
    packed = edge_index[1] * (1 << SHIFT) + edge_index[0]
    pidx = jnp.arange(EPAD - E, dtype=jnp.int32)
    pad_edges = (TRASH + pidx % (NPAD - N)) * (1 << SHIFT) + pidx % N
    edges = jnp.concatenate([packed, pad_edges])
    zeros = jnp.zeros((NPAD, D), jnp.float32)

    t00, t10 = _mm2(x, W1_00, W1_10)
    a00 = _spmm(t00, edges, zeros)
    a10 = _spmm(t10, edges, zeros)
    t01 = _relu_mm(a00, b1_00, W2_00)
    t11 = _relu_mm(a10, b1_10, W2_10)
    q00 = _spmm(t01, edges, zeros)
    q10 = _spmm(t11, edges, zeros)

    t20 = _norm_mm(q10, b2_10, W1_11)
    a20 = _spmm(t20, edges, zeros)
    t21 = _relu_mm(a20, b1_11, W2_11)
    q11 = _spmm(t21, edges, zeros)

    return _final(x, q00, b2_00, q11, b2_11)
